# baseline (speedup 1.0000x reference)
import functools

import jax
import jax.numpy as jnp
from jax import lax
from jax.experimental import pallas as pl
from jax.experimental.pallas import tpu as pltpu
from jax.experimental.pallas import tpu_sc as plsc

NC = 2
NS = 16
NW = NC * NS


def _make_sc_agg(n_nodes, n_edges, h_dim, chunk):
    epw = n_edges // NW
    nsteps = epw // chunk
    assert nsteps * chunk == epw
    rpt = (n_nodes // NS) // 8 * 8
    rem = n_nodes - NS * rpt
    assert rem % 8 == 0 and rem >= 0

    mesh = plsc.VectorSubcoreMesh(core_axis_name="c", subcore_axis_name="s")

    @functools.partial(
        pl.kernel,
        out_type=jax.ShapeDtypeStruct((NC, n_nodes, h_dim), jnp.float32),
        mesh=mesh,
        scratch_types=[
            pltpu.VMEM((epw,), jnp.int32),
            pltpu.VMEM((nsteps, chunk), jnp.int32),
            pltpu.VMEM((chunk, h_dim), jnp.float32),
            pltpu.VMEM((chunk, h_dim), jnp.float32),
            pltpu.VMEM_SHARED((n_nodes, h_dim), jnp.float32),
            pltpu.SemaphoreType.DMA,
            pltpu.SemaphoreType.DMA,
        ],
    )
    def sc_agg(h_hbm, src_hbm, dst_hbm, zero_hbm, out_hbm,
               sidx, didx, rows0, rows1, acc, gsem0, gsem1):
        c = lax.axis_index("c")
        s = lax.axis_index("s")
        w = c * NS + s
        pltpu.sync_copy(src_hbm.at[w], sidx)
        pltpu.sync_copy(dst_hbm.at[w], didx)

        assert nsteps % 2 == 1 and nsteps >= 3

        def sl(j):
            return sidx.at[pl.ds(j * chunk, chunk)]

        def gather(j, rows, sem):
            pltpu.async_copy(h_hbm.at[sl(j)], rows, sem)

        def gwait(j, rows, sem):
            pltpu.make_async_copy(h_hbm.at[sl(j)], rows, sem).wait()

        def scatter_sync(j, rows):
            pltpu.sync_copy(rows, acc.at[didx.at[j]], add=True)

        gather(0, rows0, gsem0)
        pltpu.sync_copy(zero_hbm.at[pl.ds(0, rpt)], acc.at[pl.ds(s * rpt, rpt)])
        if rem:
            @pl.when(s == NS - 1)
            def _():
                pltpu.sync_copy(zero_hbm.at[pl.ds(0, rem)],
                                acc.at[pl.ds(NS * rpt, rem)])
        plsc.subcore_barrier()

        def pair(t, _):
            j0 = 2 * t
            gather(j0 + 1, rows1, gsem1)
            gwait(j0, rows0, gsem0)
            scatter_sync(j0, rows0)
            gather(j0 + 2, rows0, gsem0)
            gwait(j0 + 1, rows1, gsem1)
            scatter_sync(j0 + 1, rows1)
            return 0

        lax.fori_loop(0, (nsteps - 1) // 2, pair, 0)
        gwait(nsteps - 1, rows0, gsem0)
        scatter_sync(nsteps - 1, rows0)
        plsc.subcore_barrier()
        pltpu.sync_copy(acc.at[pl.ds(s * rpt, rpt)],
                        out_hbm.at[c, pl.ds(s * rpt, rpt)])
        if rem:
            @pl.when(s == NS - 1)
            def _():
                pltpu.sync_copy(acc.at[pl.ds(NS * rpt, rem)],
                                out_hbm.at[c, pl.ds(NS * rpt, rem)])

    return sc_agg


def _dot(a, b):
    return jnp.dot(a.astype(jnp.bfloat16), b.astype(jnp.bfloat16),
                   preferred_element_type=jnp.float32)


def _tc_layer_body(relu, p_ref, h_ref, wr_ref, wroot_ref, br_ref, out_ref):
    agg = p_ref[0] + p_ref[1]
    o = _dot(agg, wr_ref[...]) + _dot(h_ref[...], wroot_ref[...]) + br_ref[...]
    out_ref[...] = jnp.maximum(o, 0.0) if relu else o


def _tc_layer(p, h, wr, wroot, br, relu, bn=1000):
    n, d = h.shape
    grid = n // bn
    return pl.pallas_call(
        functools.partial(_tc_layer_body, relu),
        grid=(grid,),
        in_specs=[
            pl.BlockSpec((2, bn, d), lambda i: (0, i, 0)),
            pl.BlockSpec((bn, d), lambda i: (i, 0)),
            pl.BlockSpec((d, d), lambda i: (0, 0)),
            pl.BlockSpec((d, d), lambda i: (0, 0)),
            pl.BlockSpec((1, d), lambda i: (0, 0)),
        ],
        out_specs=pl.BlockSpec((bn, d), lambda i: (i, 0)),
        out_shape=jax.ShapeDtypeStruct((n, d), jnp.float32),
    )(p, h, wr, wroot, br.reshape(1, d))


def _tc_final_body(nb, grid, p_ref, h_ref, wr_ref, wroot_ref, br_ref,
                   batch_ref, w1_ref, b1_ref, w2_ref, b2_ref,
                   out_ref, pooled_ref):
    i = pl.program_id(0)
    agg = p_ref[0] + p_ref[1]
    h3 = _dot(agg, wr_ref[...]) + _dot(h_ref[...], wroot_ref[...]) + br_ref[...]
    b = batch_ref[0]
    neg = jnp.full(h3.shape, -jnp.inf, jnp.float32)

    @pl.when(i == 0)
    def _():
        pooled_ref[...] = jnp.full_like(pooled_ref, -jnp.inf)

    for seg in range(nb):
        m = b == seg
        v = jnp.max(jnp.where(m, h3, neg), axis=0)
        pooled_ref[seg, :] = jnp.maximum(pooled_ref[seg, :], v)

    @pl.when(i == grid - 1)
    def _():
        po = pooled_ref[...]
        z = jnp.maximum(_dot(po, w1_ref[...]) + b1_ref[...], 0.0)
        out_ref[...] = _dot(z, w2_ref[...]) + b2_ref[...]


def _tc_final(p, h, wr, wroot, br, batch, w1, b1, w2, b2, nb, bn=1000):
    n, d = h.shape
    grid = n // bn
    k = w1.shape[1]
    batch3 = batch.reshape(grid, bn, 1)
    return pl.pallas_call(
        functools.partial(_tc_final_body, nb, grid),
        grid=(grid,),
        in_specs=[
            pl.BlockSpec((2, bn, d), lambda i: (0, i, 0)),
            pl.BlockSpec((bn, d), lambda i: (i, 0)),
            pl.BlockSpec((d, d), lambda i: (0, 0)),
            pl.BlockSpec((d, d), lambda i: (0, 0)),
            pl.BlockSpec((1, d), lambda i: (0, 0)),
            pl.BlockSpec((1, bn, 1), lambda i: (i, 0, 0)),
            pl.BlockSpec((d, k), lambda i: (0, 0)),
            pl.BlockSpec((1, k), lambda i: (0, 0)),
            pl.BlockSpec((k, 1), lambda i: (0, 0)),
            pl.BlockSpec((1, 1), lambda i: (0, 0)),
        ],
        out_specs=pl.BlockSpec((nb, 1), lambda i: (0, 0)),
        out_shape=jax.ShapeDtypeStruct((nb, 1), jnp.float32),
        scratch_shapes=[pltpu.VMEM((nb, d), jnp.float32)],
    )(p, h, wr, wroot, br.reshape(1, d), batch3,
      w1, b1.reshape(1, k), w2, b2.reshape(1, 1))


def kernel(x, edge_index, batch, Wr1, br1, Wroot1, Wr2, br2, Wroot2,
           Wr3, br3, Wroot3, W1, b1, W2, b2):
    n, d = x.shape
    e = edge_index.shape[1]
    nb = 16
    chunk = 80
    nsteps = e // NW // chunk
    src = edge_index[0].reshape(NW, nsteps * chunk)
    dst = edge_index[1].reshape(NW, nsteps, chunk)
    zero = jnp.zeros(((n // NS) // 8 * 8, d), jnp.float32)

    sc_agg = _make_sc_agg(n, e, d, chunk=chunk)

    p1 = sc_agg(x, src, dst, zero)
    h1 = _tc_layer(p1, x, Wr1, Wroot1, br1, relu=True)
    p2 = sc_agg(h1, src, dst, zero)
    h2 = _tc_layer(p2, h1, Wr2, Wroot2, br2, relu=True)
    p3 = sc_agg(h2, src, dst, zero)
    out = _tc_final(p3, h2, Wr3, Wroot3, br3, batch, W1, b1, W2, b2, nb)
    return out


# speedup vs baseline: 1.0139x; 1.0139x over previous
import functools

import jax
import jax.numpy as jnp
from jax import lax
from jax.experimental import pallas as pl
from jax.experimental.pallas import tpu as pltpu
from jax.experimental.pallas import tpu_sc as plsc

NC = 2
NS = 16
NW = NC * NS


def _make_sc_agg(n_nodes, n_edges, h_dim, chunk):
    epw = n_edges // NW
    nsteps = epw // chunk
    assert nsteps * chunk == epw
    rpt = (n_nodes // NS) // 8 * 8
    rem = n_nodes - NS * rpt
    assert rem % 8 == 0 and rem >= 0

    mesh = plsc.VectorSubcoreMesh(core_axis_name="c", subcore_axis_name="s")

    @functools.partial(
        pl.kernel,
        out_type=jax.ShapeDtypeStruct((NC, n_nodes, h_dim), jnp.float32),
        mesh=mesh,
        scratch_types=[
            pltpu.VMEM((epw,), jnp.int32),
            pltpu.VMEM((nsteps, chunk), jnp.int32),
            pltpu.VMEM((chunk, h_dim), jnp.float32),
            pltpu.VMEM((chunk, h_dim), jnp.float32),
            pltpu.VMEM_SHARED((n_nodes, h_dim), jnp.float32),
            pltpu.SemaphoreType.DMA,
            pltpu.SemaphoreType.DMA,
        ],
    )
    def sc_agg(h_hbm, src_hbm, dst_hbm, zero_hbm, out_hbm,
               sidx, didx, rows0, rows1, acc, gsem0, gsem1):
        c = lax.axis_index("c")
        s = lax.axis_index("s")
        w = c * NS + s
        pltpu.sync_copy(src_hbm.at[w], sidx)
        pltpu.sync_copy(dst_hbm.at[w], didx)

        assert nsteps % 2 == 1 and nsteps >= 3

        def sl(j):
            return sidx.at[pl.ds(j * chunk, chunk)]

        def gather(j, rows, sem):
            pltpu.async_copy(h_hbm.at[sl(j)], rows, sem)

        def gwait(j, rows, sem):
            pltpu.make_async_copy(h_hbm.at[sl(j)], rows, sem).wait()

        def scatter_sync(j, rows):
            pltpu.sync_copy(rows, acc.at[didx.at[j]], add=True)

        gather(0, rows0, gsem0)
        pltpu.sync_copy(zero_hbm.at[pl.ds(0, rpt)], acc.at[pl.ds(s * rpt, rpt)])
        if rem:
            @pl.when(s == NS - 1)
            def _():
                pltpu.sync_copy(zero_hbm.at[pl.ds(0, rem)],
                                acc.at[pl.ds(NS * rpt, rem)])
        plsc.subcore_barrier()

        def pair(t, _):
            j0 = 2 * t
            gather(j0 + 1, rows1, gsem1)
            gwait(j0, rows0, gsem0)
            scatter_sync(j0, rows0)
            gather(j0 + 2, rows0, gsem0)
            gwait(j0 + 1, rows1, gsem1)
            scatter_sync(j0 + 1, rows1)
            return 0

        lax.fori_loop(0, (nsteps - 1) // 2, pair, 0)
        gwait(nsteps - 1, rows0, gsem0)
        scatter_sync(nsteps - 1, rows0)
        plsc.subcore_barrier()
        pltpu.sync_copy(acc.at[pl.ds(s * rpt, rpt)],
                        out_hbm.at[c, pl.ds(s * rpt, rpt)])
        if rem:
            @pl.when(s == NS - 1)
            def _():
                pltpu.sync_copy(acc.at[pl.ds(NS * rpt, rem)],
                                out_hbm.at[c, pl.ds(NS * rpt, rem)])

    return sc_agg


def _dot(a, b):
    return jnp.dot(a.astype(jnp.bfloat16), b.astype(jnp.bfloat16),
                   preferred_element_type=jnp.float32)


def _tc_layer_body(relu, p_ref, h_ref, wr_ref, wroot_ref, br_ref, out_ref):
    agg = p_ref[0] + p_ref[1]
    o = _dot(agg, wr_ref[...]) + _dot(h_ref[...], wroot_ref[...]) + br_ref[...]
    out_ref[...] = jnp.maximum(o, 0.0) if relu else o


def _tc_layer(p, h, wr, wroot, br, relu, bn=2000):
    n, d = h.shape
    grid = n // bn
    return pl.pallas_call(
        functools.partial(_tc_layer_body, relu),
        grid=(grid,),
        in_specs=[
            pl.BlockSpec((2, bn, d), lambda i: (0, i, 0)),
            pl.BlockSpec((bn, d), lambda i: (i, 0)),
            pl.BlockSpec((d, d), lambda i: (0, 0)),
            pl.BlockSpec((d, d), lambda i: (0, 0)),
            pl.BlockSpec((1, d), lambda i: (0, 0)),
        ],
        out_specs=pl.BlockSpec((bn, d), lambda i: (i, 0)),
        out_shape=jax.ShapeDtypeStruct((n, d), jnp.float32),
    )(p, h, wr, wroot, br.reshape(1, d))


def _tc_final_body(nb, grid, p_ref, h_ref, wr_ref, wroot_ref, br_ref,
                   batch_ref, w1_ref, b1_ref, w2_ref, b2_ref,
                   out_ref, pooled_ref):
    i = pl.program_id(0)
    agg = p_ref[0] + p_ref[1]
    h3 = _dot(agg, wr_ref[...]) + _dot(h_ref[...], wroot_ref[...]) + br_ref[...]
    b = batch_ref[0]
    neg = jnp.full(h3.shape, -jnp.inf, jnp.float32)

    @pl.when(i == 0)
    def _():
        pooled_ref[...] = jnp.full_like(pooled_ref, -jnp.inf)

    for seg in range(nb):
        m = b == seg
        v = jnp.max(jnp.where(m, h3, neg), axis=0)
        pooled_ref[seg, :] = jnp.maximum(pooled_ref[seg, :], v)

    @pl.when(i == grid - 1)
    def _():
        po = pooled_ref[...]
        z = jnp.maximum(_dot(po, w1_ref[...]) + b1_ref[...], 0.0)
        out_ref[...] = _dot(z, w2_ref[...]) + b2_ref[...]


def _tc_final(p, h, wr, wroot, br, batch, w1, b1, w2, b2, nb, bn=2000):
    n, d = h.shape
    grid = n // bn
    k = w1.shape[1]
    batch3 = batch.reshape(grid, bn, 1)
    return pl.pallas_call(
        functools.partial(_tc_final_body, nb, grid),
        grid=(grid,),
        in_specs=[
            pl.BlockSpec((2, bn, d), lambda i: (0, i, 0)),
            pl.BlockSpec((bn, d), lambda i: (i, 0)),
            pl.BlockSpec((d, d), lambda i: (0, 0)),
            pl.BlockSpec((d, d), lambda i: (0, 0)),
            pl.BlockSpec((1, d), lambda i: (0, 0)),
            pl.BlockSpec((1, bn, 1), lambda i: (i, 0, 0)),
            pl.BlockSpec((d, k), lambda i: (0, 0)),
            pl.BlockSpec((1, k), lambda i: (0, 0)),
            pl.BlockSpec((k, 1), lambda i: (0, 0)),
            pl.BlockSpec((1, 1), lambda i: (0, 0)),
        ],
        out_specs=pl.BlockSpec((nb, 1), lambda i: (0, 0)),
        out_shape=jax.ShapeDtypeStruct((nb, 1), jnp.float32),
        scratch_shapes=[pltpu.VMEM((nb, d), jnp.float32)],
    )(p, h, wr, wroot, br.reshape(1, d), batch3,
      w1, b1.reshape(1, k), w2, b2.reshape(1, 1))


def kernel(x, edge_index, batch, Wr1, br1, Wroot1, Wr2, br2, Wroot2,
           Wr3, br3, Wroot3, W1, b1, W2, b2):
    n, d = x.shape
    e = edge_index.shape[1]
    nb = 16
    chunk = 80
    nsteps = e // NW // chunk
    src = edge_index[0].reshape(NW, nsteps * chunk)
    dst = edge_index[1].reshape(NW, nsteps, chunk)
    zero = jnp.zeros(((n // NS) // 8 * 8, d), jnp.float32)

    sc_agg = _make_sc_agg(n, e, d, chunk=chunk)

    p1 = sc_agg(x, src, dst, zero)
    h1 = _tc_layer(p1, x, Wr1, Wroot1, br1, relu=True)
    p2 = sc_agg(h1, src, dst, zero)
    h2 = _tc_layer(p2, h1, Wr2, Wroot2, br2, relu=True)
    p3 = sc_agg(h2, src, dst, zero)
    out = _tc_final(p3, h2, Wr3, Wroot3, br3, batch, W1, b1, W2, b2, nb)
    return out
